# baseline (speedup 1.0000x reference)
import functools

import jax
import jax.numpy as jnp
from jax import lax
from jax.experimental import pallas as pl
from jax.experimental.pallas import tpu as pltpu
from jax.experimental.pallas import tpu_sc as plsc

N = 10000
E = 320000
D = 128
WP = 144
NC = 2
NS = 16
NW = NC * NS
EPW = E // NW
CH = 80
NCH = EPW // CH
RPT = N // NS

RB = 1000
NRB = N // RB

NB = 4
NI = 8
DZR = 125
DW = 8


def _make_seg_sum(with_deg):
  width = D
  mesh = plsc.VectorSubcoreMesh(core_axis_name="c", subcore_axis_name="s",
                                num_cores=NC, num_subcores=NS)

  out_type = [jax.ShapeDtypeStruct((N, width), jnp.float32),
              jax.ShapeDtypeStruct((N, width), jnp.float32)]
  scratch = [pltpu.VMEM_SHARED((N, width), jnp.float32)]
  scratch += [pltpu.VMEM((CH,), jnp.int32) for _ in range(NI)]
  scratch += [pltpu.VMEM((CH,), jnp.int32) for _ in range(NI)]
  scratch += [pltpu.VMEM((CH, width), jnp.float32) for _ in range(NB)]
  scratch += [pltpu.SemaphoreType.DMA for _ in range(NB)]
  scratch += [pltpu.SemaphoreType.DMA for _ in range(NI)]
  scratch += [pltpu.SemaphoreType.DMA for _ in range(NB)]
  if with_deg:
    out_type += [jax.ShapeDtypeStruct((N, DW), jnp.float32),
                 jax.ShapeDtypeStruct((N, DW), jnp.float32)]
    scratch += [
        pltpu.VMEM_SHARED((N, DW), jnp.float32),
        pltpu.VMEM((CH, DW), jnp.float32),
        pltpu.VMEM((DZR, DW), jnp.float32),
    ]
    scratch += [pltpu.SemaphoreType.DMA for _ in range(NB)]

  @functools.partial(
      pl.kernel,
      out_type=tuple(out_type),
      mesh=mesh,
      scratch_types=tuple(scratch),
      compiler_params=pltpu.CompilerParams(use_tc_tiling_on_sc=False),
  )
  def seg(rows_hbm, eidx_hbm, *rest):
    if with_deg:
      ones_hbm, dz_hbm = rest[:2]
      rest = rest[2:]
      out0, out1, dout0, dout1, acc = rest[:5]
      rest = rest[5:]
    else:
      out0, out1, acc = rest[:3]
      rest = rest[3:]
    sbufs = rest[:NI]
    dbufs = rest[NI:2 * NI]
    rest = rest[2 * NI:]
    rbufs = rest[:NB]
    gsems = rest[NB:2 * NB]
    isems = rest[2 * NB:2 * NB + NI]
    ssems = rest[2 * NB + NI:3 * NB + NI]
    rest = rest[3 * NB + NI:]
    if with_deg:
      dacc, ones_v, dz = rest[:3]
      dsems = rest[3:]
    cid = lax.axis_index("c")
    sid = lax.axis_index("s")
    wid = sid * NC + cid

    def start_i(c, q):
      base = wid * EPW + c * CH
      pltpu.async_copy(eidx_hbm.at[0, pl.ds(base, CH)], sbufs[q], isems[q])
      pltpu.async_copy(eidx_hbm.at[1, pl.ds(base, CH)], dbufs[q], isems[q])

    def wait_i(c, q):
      base = wid * EPW + c * CH
      pltpu.make_async_copy(eidx_hbm.at[0, pl.ds(base, CH)], sbufs[q], isems[q]).wait()
      pltpu.make_async_copy(eidx_hbm.at[1, pl.ds(base, CH)], dbufs[q], isems[q]).wait()

    def start_g(p, q):
      pltpu.async_copy(rows_hbm.at[sbufs[q]], rbufs[p], gsems[p])

    def wait_g(p, q):
      pltpu.make_async_copy(rows_hbm.at[sbufs[q]], rbufs[p], gsems[p]).wait()

    def start_s(p, q):
      pltpu.async_copy(rbufs[p], acc.at[dbufs[q]], ssems[p], add=True)
      if with_deg:
        pltpu.async_copy(ones_v, dacc.at[dbufs[q]], dsems[p], add=True)

    def wait_s(p, q):
      pltpu.make_async_copy(rbufs[p], acc.at[dbufs[q]], ssems[p]).wait()
      if with_deg:
        pltpu.make_async_copy(ones_v, dacc.at[dbufs[q]], dsems[p]).wait()

    for k in range(6):
      start_i(k, k)
    wait_i(0, 0)
    start_g(0, 0)
    wait_i(1, 1)
    start_g(1, 1)

    zbuf = rbufs[NB - 1]
    cpr = width // 16
    def zb(i, _):
      r = i // cpr
      col = (i % cpr) * 16
      zbuf[r, pl.ds(col, 16)] = jnp.zeros((16,), jnp.float32)
      return 0
    lax.fori_loop(0, CH * cpr, zb, 0)

    nfull = RPT // CH
    def zcp(j, _):
      pltpu.sync_copy(zbuf, acc.at[pl.ds(sid * RPT + j * CH, CH)])
      return 0
    lax.fori_loop(0, nfull, zcp, 0)
    rem = RPT - nfull * CH
    if rem:
      pltpu.sync_copy(zbuf.at[pl.ds(0, rem)],
                      acc.at[pl.ds(sid * RPT + nfull * CH, rem)])

    if with_deg:
      pltpu.sync_copy(ones_hbm, ones_v)
      pltpu.sync_copy(dz_hbm, dz)
      def dzcp(j, _):
        pltpu.sync_copy(dz, dacc.at[pl.ds(sid * RPT + j * DZR, DZR)])
        return 0
      lax.fori_loop(0, RPT // DZR, dzcp, 0)
    plsc.subcore_barrier()

    def chunk(c, _):
      def stage(q):
        p = q % NB
        wait_g(p, q)
        @pl.when(c >= 2)
        def _():
          wait_s((p + 2) % NB, (q + 6) % NI)
        @pl.when(c + 2 < NCH)
        def _():
          wait_i(c + 2, (q + 2) % NI)
          start_g((p + 2) % NB, (q + 2) % NI)
        start_s(p, q)
        @pl.when(c + 6 < NCH)
        def _():
          start_i(c + 6, (q + 6) % NI)

      for q in range(NI):
        @pl.when(c % NI == q)
        def _(q=q):
          stage(q)
      return 0
    lax.fori_loop(0, NCH, chunk, 0)

    wait_s((NCH - 2) % NB, (NCH - 2) % NI)
    wait_s((NCH - 1) % NB, (NCH - 1) % NI)
    plsc.subcore_barrier()

    row0 = sid * RPT

    @pl.when(cid == 0)
    def _():
      pltpu.sync_copy(acc.at[pl.ds(row0, RPT)], out0.at[pl.ds(row0, RPT)])
      if with_deg:
        pltpu.sync_copy(dacc.at[pl.ds(row0, RPT)], dout0.at[pl.ds(row0, RPT)])

    @pl.when(cid == 1)
    def _():
      pltpu.sync_copy(acc.at[pl.ds(row0, RPT)], out1.at[pl.ds(row0, RPT)])
      if with_deg:
        pltpu.sync_copy(dacc.at[pl.ds(row0, RPT)], dout1.at[pl.ds(row0, RPT)])

  return seg


_seg_sum_l1 = _make_seg_sum(True)
_seg_sum_l2 = _make_seg_sum(False)


def _dot_t(a, w):
  return lax.dot_general(a, w, (((1,), (1,)), ((), ())),
                         preferred_element_type=jnp.float32,
                         precision=lax.Precision.HIGHEST)


def _tcr_body(x_ref, w_ref, b_ref, o_ref):
  o_ref[...] = _dot_t(x_ref[...], w_ref[...]) + b_ref[...]


def _tcr(x, w, b):
  return pl.pallas_call(
      _tcr_body,
      grid=(NRB,),
      in_specs=[
          pl.BlockSpec((RB, D), lambda i: (i, 0)),
          pl.BlockSpec((D, D), lambda i: (0, 0)),
          pl.BlockSpec((1, D), lambda i: (0, 0)),
      ],
      out_specs=pl.BlockSpec((RB, D), lambda i: (i, 0)),
      out_shape=jax.ShapeDtypeStruct((N, D), jnp.float32),
  )(x, w, b)


def _tc1_body(p0_ref, p1_ref, d0_ref, d1_ref, xr_ref, wl_ref,
              hpre_ref, deg_ref, stats_ref):
  deg = jnp.maximum(d0_ref[:, 0:1] + d1_ref[:, 0:1], 1.0)
  agg = (p0_ref[...] + p1_ref[...]) / deg
  hpre = _dot_t(agg, wl_ref[...]) + xr_ref[...]
  hpre_ref[...] = hpre
  deg_ref[...] = deg

  @pl.when(pl.program_id(0) == 0)
  def _():
    stats_ref[...] = jnp.zeros((8, D), jnp.float32)

  ps = jnp.sum(hpre, axis=0, keepdims=True)
  pq = jnp.sum(hpre * hpre, axis=0, keepdims=True)
  stats_ref[...] += jnp.concatenate(
      [ps, pq, jnp.zeros((6, D), jnp.float32)], axis=0)


def _tc1(p0, p1, d0, d1, xr, W1_l):
  return pl.pallas_call(
      _tc1_body,
      grid=(NRB,),
      in_specs=[
          pl.BlockSpec((RB, D), lambda i: (i, 0)),
          pl.BlockSpec((RB, D), lambda i: (i, 0)),
          pl.BlockSpec((RB, DW), lambda i: (i, 0)),
          pl.BlockSpec((RB, DW), lambda i: (i, 0)),
          pl.BlockSpec((RB, D), lambda i: (i, 0)),
          pl.BlockSpec((D, D), lambda i: (0, 0)),
      ],
      out_specs=[
          pl.BlockSpec((RB, D), lambda i: (i, 0)),
          pl.BlockSpec((RB, 1), lambda i: (i, 0)),
          pl.BlockSpec((8, D), lambda i: (0, 0)),
      ],
      out_shape=[
          jax.ShapeDtypeStruct((N, D), jnp.float32),
          jax.ShapeDtypeStruct((N, 1), jnp.float32),
          jax.ShapeDtypeStruct((8, D), jnp.float32),
      ],
  )(p0, p1, d0, d1, xr, W1_l)


def _tc_norm_body(hpre_ref, stats_ref, gamma_ref, beta_ref, w2r_ref, b2_ref,
                  h_ref, hr_ref):
  s = stats_ref[...]
  mean = s[0:1, :] / N
  var = s[1:2, :] / N - mean * mean
  inv = lax.rsqrt(var + 1e-5)
  hn = (hpre_ref[...] - mean) * inv * gamma_ref[...] + beta_ref[...]
  h = jnp.maximum(hn, 0.0)
  h_ref[...] = h
  hr_ref[...] = _dot_t(h, w2r_ref[...]) + b2_ref[...]


def _tc_norm(hpre, stats, gamma, beta, W2_r, b2):
  return pl.pallas_call(
      _tc_norm_body,
      grid=(NRB,),
      in_specs=[
          pl.BlockSpec((RB, D), lambda i: (i, 0)),
          pl.BlockSpec((8, D), lambda i: (0, 0)),
          pl.BlockSpec((1, D), lambda i: (0, 0)),
          pl.BlockSpec((1, D), lambda i: (0, 0)),
          pl.BlockSpec((D, D), lambda i: (0, 0)),
          pl.BlockSpec((1, D), lambda i: (0, 0)),
      ],
      out_specs=[
          pl.BlockSpec((RB, D), lambda i: (i, 0)),
          pl.BlockSpec((RB, D), lambda i: (i, 0)),
      ],
      out_shape=[
          jax.ShapeDtypeStruct((N, D), jnp.float32),
          jax.ShapeDtypeStruct((N, D), jnp.float32),
      ],
  )(hpre, stats, gamma, beta, W2_r, b2)


def _tc2_body(q0_ref, q1_ref, deg_ref, hr_ref, wl_ref, out_ref):
  agg = (q0_ref[...] + q1_ref[...]) / deg_ref[...]
  out_ref[...] = _dot_t(agg, wl_ref[...]) + hr_ref[...]


def _tc2(q0, q1, deg, hr, W2_l):
  return pl.pallas_call(
      _tc2_body,
      grid=(NRB,),
      in_specs=[
          pl.BlockSpec((RB, D), lambda i: (i, 0)),
          pl.BlockSpec((RB, D), lambda i: (i, 0)),
          pl.BlockSpec((RB, 1), lambda i: (i, 0)),
          pl.BlockSpec((RB, D), lambda i: (i, 0)),
          pl.BlockSpec((D, D), lambda i: (0, 0)),
      ],
      out_specs=pl.BlockSpec((RB, D), lambda i: (i, 0)),
      out_shape=jax.ShapeDtypeStruct((N, D), jnp.float32),
  )(q0, q1, deg, hr, W2_l)


def kernel(x, edge_index, W1_l, b1_l, W1_r, gamma, beta, W2_l, b2_l, W2_r):
  ones_s = jnp.ones((CH, DW), jnp.float32)
  dz_s = jnp.zeros((DZR, DW), jnp.float32)
  p0, p1, d0, d1 = _seg_sum_l1(x, edge_index, ones_s, dz_s)
  xr = _tcr(x, W1_r, b1_l.reshape(1, D))
  hpre, deg, stats = _tc1(p0, p1, d0, d1, xr, W1_l)
  h, hr = _tc_norm(hpre, stats, gamma.reshape(1, D), beta.reshape(1, D),
                   W2_r, b2_l.reshape(1, D))
  q0, q1 = _seg_sum_l2(h, edge_index)
  return _tc2(q0, q1, deg, hr, W2_l)


# speedup vs baseline: 1.0157x; 1.0157x over previous
import functools

import jax
import jax.numpy as jnp
from jax import lax
from jax.experimental import pallas as pl
from jax.experimental.pallas import tpu as pltpu
from jax.experimental.pallas import tpu_sc as plsc

N = 10000
E = 320000
D = 128
WP = 144
NC = 2
NS = 16
NW = NC * NS
EPW = E // NW
CH = 80
NCH = EPW // CH
RPT = N // NS

RB = 1000
NRB = N // RB

NB = 4
NI = 8
DZR = 125
DW = 8


def _make_seg_sum(with_deg):
  width = D
  mesh = plsc.VectorSubcoreMesh(core_axis_name="c", subcore_axis_name="s",
                                num_cores=NC, num_subcores=NS)

  out_type = [jax.ShapeDtypeStruct((N, width), jnp.float32),
              jax.ShapeDtypeStruct((N, width), jnp.float32)]
  scratch = [pltpu.VMEM_SHARED((N, width), jnp.float32)]
  scratch += [pltpu.VMEM((CH,), jnp.int32) for _ in range(NI)]
  scratch += [pltpu.VMEM((CH,), jnp.int32) for _ in range(NI)]
  scratch += [pltpu.VMEM((CH, width), jnp.float32) for _ in range(NB)]
  scratch += [pltpu.SemaphoreType.DMA for _ in range(NB)]
  scratch += [pltpu.SemaphoreType.DMA for _ in range(NI)]
  scratch += [pltpu.SemaphoreType.DMA for _ in range(NB)]
  if with_deg:
    out_type += [jax.ShapeDtypeStruct((N, DW), jnp.float32),
                 jax.ShapeDtypeStruct((N, DW), jnp.float32)]
    scratch += [
        pltpu.VMEM_SHARED((N, DW), jnp.float32),
        pltpu.VMEM((CH, DW), jnp.float32),
        pltpu.VMEM((DZR, DW), jnp.float32),
    ]
    scratch += [pltpu.SemaphoreType.DMA for _ in range(NB)]

  @functools.partial(
      pl.kernel,
      out_type=tuple(out_type),
      mesh=mesh,
      scratch_types=tuple(scratch),
      compiler_params=pltpu.CompilerParams(use_tc_tiling_on_sc=False),
  )
  def seg(rows_hbm, eidx_hbm, *rest):
    if with_deg:
      ones_hbm, dz_hbm = rest[:2]
      rest = rest[2:]
      out0, out1, dout0, dout1, acc = rest[:5]
      rest = rest[5:]
    else:
      out0, out1, acc = rest[:3]
      rest = rest[3:]
    sbufs = rest[:NI]
    dbufs = rest[NI:2 * NI]
    rest = rest[2 * NI:]
    rbufs = rest[:NB]
    gsems = rest[NB:2 * NB]
    isems = rest[2 * NB:2 * NB + NI]
    ssems = rest[2 * NB + NI:3 * NB + NI]
    rest = rest[3 * NB + NI:]
    if with_deg:
      dacc, ones_v, dz = rest[:3]
      dsems = rest[3:]
    cid = lax.axis_index("c")
    sid = lax.axis_index("s")
    wid = sid * NC + cid

    def start_i(c, q):
      base = wid * EPW + c * CH
      pltpu.async_copy(eidx_hbm.at[0, pl.ds(base, CH)], sbufs[q], isems[q])
      pltpu.async_copy(eidx_hbm.at[1, pl.ds(base, CH)], dbufs[q], isems[q])

    def wait_i(c, q):
      base = wid * EPW + c * CH
      pltpu.make_async_copy(eidx_hbm.at[0, pl.ds(base, CH)], sbufs[q], isems[q]).wait()
      pltpu.make_async_copy(eidx_hbm.at[1, pl.ds(base, CH)], dbufs[q], isems[q]).wait()

    def start_g(p, q):
      pltpu.async_copy(rows_hbm.at[sbufs[q]], rbufs[p], gsems[p])

    def wait_g(p, q):
      pltpu.make_async_copy(rows_hbm.at[sbufs[q]], rbufs[p], gsems[p]).wait()

    def start_s(p, q):
      pltpu.async_copy(rbufs[p], acc.at[dbufs[q]], ssems[p], add=True)
      if with_deg:
        pltpu.async_copy(ones_v, dacc.at[dbufs[q]], dsems[p], add=True)

    def wait_s(p, q):
      pltpu.make_async_copy(rbufs[p], acc.at[dbufs[q]], ssems[p]).wait()
      if with_deg:
        pltpu.make_async_copy(ones_v, dacc.at[dbufs[q]], dsems[p]).wait()

    for k in range(6):
      start_i(k, k)
    wait_i(0, 0)
    start_g(0, 0)
    wait_i(1, 1)
    start_g(1, 1)

    zbuf = rbufs[NB - 1]
    cpr = width // 16
    def zb(i, _):
      r = i // cpr
      col = (i % cpr) * 16
      zbuf[r, pl.ds(col, 16)] = jnp.zeros((16,), jnp.float32)
      return 0
    lax.fori_loop(0, CH * cpr, zb, 0)

    nfull = RPT // CH
    def zcp(j, _):
      pltpu.sync_copy(zbuf, acc.at[pl.ds(sid * RPT + j * CH, CH)])
      return 0
    lax.fori_loop(0, nfull, zcp, 0)
    rem = RPT - nfull * CH
    if rem:
      pltpu.sync_copy(zbuf.at[pl.ds(0, rem)],
                      acc.at[pl.ds(sid * RPT + nfull * CH, rem)])

    if with_deg:
      pltpu.sync_copy(ones_hbm, ones_v)
      pltpu.sync_copy(dz_hbm, dz)
      def dzcp(j, _):
        pltpu.sync_copy(dz, dacc.at[pl.ds(sid * RPT + j * DZR, DZR)])
        return 0
      lax.fori_loop(0, RPT // DZR, dzcp, 0)
    plsc.subcore_barrier()

    def chunk(c, _):
      def stage(q):
        p = q % NB
        wait_g(p, q)
        @pl.when(c >= 2)
        def _():
          wait_s((p + 2) % NB, (q + 6) % NI)
        @pl.when(c + 2 < NCH)
        def _():
          wait_i(c + 2, (q + 2) % NI)
          start_g((p + 2) % NB, (q + 2) % NI)
        start_s(p, q)
        @pl.when(c + 6 < NCH)
        def _():
          start_i(c + 6, (q + 6) % NI)

      for q in range(NI):
        @pl.when(c % NI == q)
        def _(q=q):
          stage(q)
      return 0
    lax.fori_loop(0, NCH, chunk, 0)

    wait_s((NCH - 2) % NB, (NCH - 2) % NI)
    wait_s((NCH - 1) % NB, (NCH - 1) % NI)
    plsc.subcore_barrier()

    row0 = sid * RPT

    @pl.when(cid == 0)
    def _():
      pltpu.sync_copy(acc.at[pl.ds(row0, RPT)], out0.at[pl.ds(row0, RPT)])
      if with_deg:
        pltpu.sync_copy(dacc.at[pl.ds(row0, RPT)], dout0.at[pl.ds(row0, RPT)])

    @pl.when(cid == 1)
    def _():
      pltpu.sync_copy(acc.at[pl.ds(row0, RPT)], out1.at[pl.ds(row0, RPT)])
      if with_deg:
        pltpu.sync_copy(dacc.at[pl.ds(row0, RPT)], dout1.at[pl.ds(row0, RPT)])

  return seg


_seg_sum_l1 = _make_seg_sum(True)
_seg_sum_l2 = _make_seg_sum(False)


def _dot_t(a, w):
  return lax.dot_general(a, w, (((1,), (1,)), ((), ())),
                         preferred_element_type=jnp.float32,
                         precision=lax.Precision.HIGHEST)


def _tcr_body(x_ref, w_ref, b_ref, o_ref):
  o_ref[...] = _dot_t(x_ref[...], w_ref[...]) + b_ref[...]


def _tcr(x, w, b):
  return pl.pallas_call(
      _tcr_body,
      grid=(NRB,),
      in_specs=[
          pl.BlockSpec((RB, D), lambda i: (i, 0)),
          pl.BlockSpec((D, D), lambda i: (0, 0)),
          pl.BlockSpec((1, D), lambda i: (0, 0)),
      ],
      out_specs=pl.BlockSpec((RB, D), lambda i: (i, 0)),
      out_shape=jax.ShapeDtypeStruct((N, D), jnp.float32),
  )(x, w, b)


def _tc1_body(p0_ref, p1_ref, d0_ref, d1_ref, xr_ref, wl_ref,
              hpre_ref, deg_ref, stats_ref):
  deg = jnp.maximum(d0_ref[:, 0:1] + d1_ref[:, 0:1], 1.0)
  agg = (p0_ref[...] + p1_ref[...]) / deg
  hpre = _dot_t(agg, wl_ref[...]) + xr_ref[...]
  hpre_ref[...] = hpre
  deg_ref[...] = deg

  @pl.when(pl.program_id(0) == 0)
  def _():
    stats_ref[...] = jnp.zeros((8, D), jnp.float32)

  ps = jnp.sum(hpre, axis=0, keepdims=True)
  pq = jnp.sum(hpre * hpre, axis=0, keepdims=True)
  stats_ref[...] += jnp.concatenate(
      [ps, pq, jnp.zeros((6, D), jnp.float32)], axis=0)


def _tc1(p0, p1, d0, d1, xr, W1_l):
  return pl.pallas_call(
      _tc1_body,
      grid=(NRB,),
      in_specs=[
          pl.BlockSpec((RB, D), lambda i: (i, 0)),
          pl.BlockSpec((RB, D), lambda i: (i, 0)),
          pl.BlockSpec((RB, DW), lambda i: (i, 0)),
          pl.BlockSpec((RB, DW), lambda i: (i, 0)),
          pl.BlockSpec((RB, D), lambda i: (i, 0)),
          pl.BlockSpec((D, D), lambda i: (0, 0)),
      ],
      out_specs=[
          pl.BlockSpec((RB, D), lambda i: (i, 0)),
          pl.BlockSpec((RB, 1), lambda i: (i, 0)),
          pl.BlockSpec((8, D), lambda i: (0, 0)),
      ],
      out_shape=[
          jax.ShapeDtypeStruct((N, D), jnp.float32),
          jax.ShapeDtypeStruct((N, 1), jnp.float32),
          jax.ShapeDtypeStruct((8, D), jnp.float32),
      ],
  )(p0, p1, d0, d1, xr, W1_l)


def _tc_norm_body(hpre_ref, stats_ref, gamma_ref, beta_ref, h_ref):
  s = stats_ref[...]
  mean = s[0:1, :] / N
  var = s[1:2, :] / N - mean * mean
  inv = lax.rsqrt(var + 1e-5)
  hn = (hpre_ref[...] - mean) * inv * gamma_ref[...] + beta_ref[...]
  h_ref[...] = jnp.maximum(hn, 0.0)


def _tc_norm(hpre, stats, gamma, beta):
  return pl.pallas_call(
      _tc_norm_body,
      grid=(NRB,),
      in_specs=[
          pl.BlockSpec((RB, D), lambda i: (i, 0)),
          pl.BlockSpec((8, D), lambda i: (0, 0)),
          pl.BlockSpec((1, D), lambda i: (0, 0)),
          pl.BlockSpec((1, D), lambda i: (0, 0)),
      ],
      out_specs=pl.BlockSpec((RB, D), lambda i: (i, 0)),
      out_shape=jax.ShapeDtypeStruct((N, D), jnp.float32),
  )(hpre, stats, gamma, beta)


def _tc2_body(q0_ref, q1_ref, deg_ref, hr_ref, wl_ref, out_ref):
  agg = (q0_ref[...] + q1_ref[...]) / deg_ref[...]
  out_ref[...] = _dot_t(agg, wl_ref[...]) + hr_ref[...]


def _tc2(q0, q1, deg, hr, W2_l):
  return pl.pallas_call(
      _tc2_body,
      grid=(NRB,),
      in_specs=[
          pl.BlockSpec((RB, D), lambda i: (i, 0)),
          pl.BlockSpec((RB, D), lambda i: (i, 0)),
          pl.BlockSpec((RB, 1), lambda i: (i, 0)),
          pl.BlockSpec((RB, D), lambda i: (i, 0)),
          pl.BlockSpec((D, D), lambda i: (0, 0)),
      ],
      out_specs=pl.BlockSpec((RB, D), lambda i: (i, 0)),
      out_shape=jax.ShapeDtypeStruct((N, D), jnp.float32),
  )(q0, q1, deg, hr, W2_l)


def kernel(x, edge_index, W1_l, b1_l, W1_r, gamma, beta, W2_l, b2_l, W2_r):
  ones_s = jnp.ones((CH, DW), jnp.float32)
  dz_s = jnp.zeros((DZR, DW), jnp.float32)
  p0, p1, d0, d1 = _seg_sum_l1(x, edge_index, ones_s, dz_s)
  xr = _tcr(x, W1_r, b1_l.reshape(1, D))
  hpre, deg, stats = _tc1(p0, p1, d0, d1, xr, W1_l)
  h = _tc_norm(hpre, stats, gamma.reshape(1, D), beta.reshape(1, D))
  q0, q1 = _seg_sum_l2(h, edge_index)
  hr = _tcr(h, W2_r, b2_l.reshape(1, D))
  return _tc2(q0, q1, deg, hr, W2_l)
